# baseline (speedup 1.0000x reference)
import functools

import jax
import jax.numpy as jnp
from jax import lax
from jax.experimental import pallas as pl
from jax.experimental.pallas import tpu as pltpu
from jax.experimental.pallas import tpu_sc as plsc

N_NODES = 10000
D = 128
N_EDGES = 320000

NC = 2
NS = 16
NW = NC * NS
EW = N_EDGES // NW
AK = 40
ANCHUNK = EW // AK
APH = 24
NBUF = 7
ZB = 624
ZL = N_NODES - (NS - 1) * ZB

_DN = (((1,), (1,)), ((), ()))
_PREC = None


_MESH = plsc.VectorSubcoreMesh(core_axis_name="c", subcore_axis_name="s")


def _agg_body(u_hbm, src_hbm, dst_hbm, zf_hbm, out_hbm,
              sidx, didx, rows, accum, ssem, *gsems):
    c = lax.axis_index("c")
    s = lax.axis_index("s")
    wid = s * NC + c

    off = pl.multiple_of(s * ZB, 8)

    @pl.when(s < NS - 1)
    def _():
        pltpu.sync_copy(zf_hbm.at[pl.ds(0, ZB), :],
                        accum.at[pl.ds(off, ZB), :])

    @pl.when(s == NS - 1)
    def _():
        pltpu.sync_copy(zf_hbm, accum.at[pl.ds((NS - 1) * ZB, ZL), :])

    plsc.subcore_barrier()

    def group(l0, nb):
        gds = [pltpu.async_copy(u_hbm.at[sidx.at[l0 + b]], rows.at[b],
                                gsems[b])
               for b in range(nb)]
        sds = []
        for b in range(nb):
            gds[b].wait()
            sds.append(pltpu.async_copy(rows.at[b], accum.at[didx.at[l0 + b]],
                                        ssem, add=True))
        for d in sds:
            d.wait()

    def step(i, carry):
        group(NBUF * i, NBUF)
        return carry

    for cbase in range(0, ANCHUNK, APH):
        nch = min(APH, ANCHUNK - cbase)
        pltpu.sync_copy(src_hbm.at[wid, pl.ds(cbase, nch), :],
                        sidx.at[pl.ds(0, nch), :])
        pltpu.sync_copy(dst_hbm.at[wid, pl.ds(cbase, nch), :],
                        didx.at[pl.ds(0, nch), :])
        lax.fori_loop(0, nch // NBUF, step, 0)
        if nch % NBUF:
            group(nch - nch % NBUF, nch % NBUF)

    plsc.subcore_barrier()

    @pl.when(s < NS - 1)
    def _():
        pltpu.sync_copy(accum.at[pl.ds(off, ZB), :],
                        out_hbm.at[c, pl.ds(off, ZB), :])

    @pl.when(s == NS - 1)
    def _():
        pltpu.sync_copy(accum.at[pl.ds((NS - 1) * ZB, ZL), :],
                        out_hbm.at[c, pl.ds((NS - 1) * ZB, ZL), :])


_agg = pl.kernel(
    _agg_body,
    out_type=jax.ShapeDtypeStruct((NC, N_NODES, D), jnp.float32),
    mesh=_MESH,
    scratch_types=[
        pltpu.VMEM((APH, AK), jnp.int32),
        pltpu.VMEM((APH, AK), jnp.int32),
        pltpu.VMEM((NBUF, AK, D), jnp.float32),
        pltpu.VMEM_SHARED((N_NODES, D), jnp.float32),
        pltpu.SemaphoreType.DMA,
    ] + [pltpu.SemaphoreType.DMA] * NBUF)


_HR = 80


def _counts_body(dst_hbm, zf_hbm, idt_hbm, cout_hbm, dflat, cl, idt, csh):
    c = lax.axis_index("c")
    s = lax.axis_index("s")
    wid = s * NC + c

    pltpu.sync_copy(zf_hbm.at[pl.ds(0, _HR), :], cl)
    pltpu.sync_copy(idt_hbm, idt)
    pltpu.sync_copy(dst_hbm.at[wid], dflat)

    @pl.when(s == 0)
    def _():
        pltpu.sync_copy(zf_hbm.at[pl.ds(0, _HR), :], csh)

    plsc.subcore_barrier()

    ones16 = jnp.ones((16,), jnp.float32)

    def hstep(k, carry):
        idx = dflat[pl.ds(k * 16, 16)]
        row = lax.shift_right_logical(idx, 7)
        lane = lax.bitwise_and(idx, 127)
        plsc.addupdate_scatter(cl, [row, lane], ones16)
        return carry

    lax.fori_loop(0, EW // 16, hstep, 0)
    pltpu.sync_copy(cl, csh.at[idt.at[0]], add=True)
    plsc.subcore_barrier()

    @pl.when(s == 0)
    def _():
        pltpu.sync_copy(csh, cout_hbm.at[c])


_counts = pl.kernel(
    _counts_body,
    out_type=jax.ShapeDtypeStruct((NC, _HR, D), jnp.float32),
    mesh=_MESH,
    compiler_params=pltpu.CompilerParams(needs_layout_passes=False),
    scratch_types=[
        pltpu.VMEM((EW,), jnp.int32),
        pltpu.VMEM((_HR, D), jnp.float32),
        pltpu.VMEM((1, _HR), jnp.int32),
        pltpu.VMEM_SHARED((_HR, D), jnp.float32),
    ])


_BR = 2000


def _dense_body(x_ref, wl_ref, wr_ref, b_ref, u_ref, v_ref):
    xb = x_ref[...]
    u_ref[...] = lax.dot_general(xb, wl_ref[...], _DN,
                                 preferred_element_type=jnp.float32,
                                 precision=_PREC)
    v_ref[...] = lax.dot_general(xb, wr_ref[...], _DN,
                                 preferred_element_type=jnp.float32,
                                 precision=_PREC) + b_ref[...]


def _recip_body(cnt_ref, r_ref):
    r_ref[...] = 1.0 / jnp.maximum(cnt_ref[0] + cnt_ref[1], 1.0)


def _combine_body(s_ref, rc_ref, v1_ref, wl_ref, wr_ref, b_ref,
                  u2_ref, v2_ref):
    h = jnp.maximum((s_ref[0] + s_ref[1]) * rc_ref[...] + v1_ref[...], 0.0)
    u2_ref[...] = lax.dot_general(h, wl_ref[...], _DN,
                                  preferred_element_type=jnp.float32,
                                  precision=_PREC)
    v2_ref[...] = lax.dot_general(h, wr_ref[...], _DN,
                                  preferred_element_type=jnp.float32,
                                  precision=_PREC) + b_ref[...]


def _final_body(s_ref, rc_ref, v2_ref, o_ref):
    o_ref[...] = (s_ref[0] + s_ref[1]) * rc_ref[...] + v2_ref[...]


_row_spec = pl.BlockSpec((_BR, D), lambda i: (i, 0))
_w_spec = pl.BlockSpec((D, D), lambda i: (0, 0))
_b_spec = pl.BlockSpec((1, D), lambda i: (0, 0))
_part_spec = pl.BlockSpec((NC, _BR, D), lambda i: (0, i, 0))
_rc_spec = pl.BlockSpec((_BR, 1), lambda i: (i, 0))
_GRID = (N_NODES // _BR,)
_f32 = functools.partial(jax.ShapeDtypeStruct, dtype=jnp.float32)

_dense = pl.pallas_call(
    _dense_body, grid=_GRID,
    in_specs=[_row_spec, _w_spec, _w_spec, _b_spec],
    out_specs=[_row_spec, _row_spec],
    out_shape=[_f32((N_NODES, D)), _f32((N_NODES, D))])

_recip = pl.pallas_call(
    _recip_body,
    in_specs=[pl.BlockSpec((NC, _HR, D), lambda: (0, 0, 0))],
    out_specs=pl.BlockSpec((_HR, D), lambda: (0, 0)),
    out_shape=_f32((_HR, D)))

_combine = pl.pallas_call(
    _combine_body, grid=_GRID,
    in_specs=[_part_spec, _rc_spec, _row_spec, _w_spec, _w_spec, _b_spec],
    out_specs=[_row_spec, _row_spec],
    out_shape=[_f32((N_NODES, D)), _f32((N_NODES, D))])

_final = pl.pallas_call(
    _final_body, grid=_GRID,
    in_specs=[_part_spec, _rc_spec, _row_spec],
    out_specs=_row_spec,
    out_shape=_f32((N_NODES, D)))


def kernel(x, edge_index, Wl1, bl1, Wr1, Wl2, bl2, Wr2):
    src = edge_index[0].astype(jnp.int32)
    dst = edge_index[1].astype(jnp.int32)
    src3a = src.reshape(NW, ANCHUNK, AK)
    dst3a = dst.reshape(NW, ANCHUNK, AK)
    dst2 = dst.reshape(NW, EW)
    zf = jnp.zeros((ZL, D), jnp.float32)
    idt = jnp.arange(_HR, dtype=jnp.int32).reshape(1, _HR)

    u1, v1 = _dense(x, Wl1, Wr1, bl1.reshape(1, D))
    cnt = _counts(dst2, zf, idt)
    rc = _recip(cnt).reshape(_HR * D)[:N_NODES].reshape(N_NODES, 1)
    s1 = _agg(u1, src3a, dst3a, zf)
    u2, v2 = _combine(s1, rc, v1, Wl2, Wr2, bl2.reshape(1, D))
    s2 = _agg(u2, src3a, dst3a, zf)
    return _final(s2, rc, v2)


# speedup vs baseline: 1.0519x; 1.0519x over previous
import functools

import jax
import jax.numpy as jnp
from jax import lax
from jax.experimental import pallas as pl
from jax.experimental.pallas import tpu as pltpu
from jax.experimental.pallas import tpu_sc as plsc

N_NODES = 10000
D = 128
N_EDGES = 320000

NC = 2
NS = 16
NW = NC * NS
EW = N_EDGES // NW
AK = 40
ANCHUNK = EW // AK
APH = 48
NBUF = 6
ZB = 624
ZL = N_NODES - (NS - 1) * ZB

_DN = (((1,), (1,)), ((), ()))
_PREC = None


_MESH = plsc.VectorSubcoreMesh(core_axis_name="c", subcore_axis_name="s")


def _agg_body(u_hbm, src_hbm, dst_hbm, zf_hbm, out_hbm,
              sidx, didx, rows, accum, ssem, *gsems):
    c = lax.axis_index("c")
    s = lax.axis_index("s")
    wid = s * NC + c

    off = pl.multiple_of(s * ZB, 8)

    @pl.when(s < NS - 1)
    def _():
        pltpu.sync_copy(zf_hbm.at[pl.ds(0, ZB), :],
                        accum.at[pl.ds(off, ZB), :])

    @pl.when(s == NS - 1)
    def _():
        pltpu.sync_copy(zf_hbm, accum.at[pl.ds((NS - 1) * ZB, ZL), :])

    plsc.subcore_barrier()

    def group(l0, nb):
        gds = [pltpu.async_copy(u_hbm.at[sidx.at[l0 + b]], rows.at[b],
                                gsems[b])
               for b in range(nb)]
        sds = []
        for b in range(nb):
            gds[b].wait()
            sds.append(pltpu.async_copy(rows.at[b], accum.at[didx.at[l0 + b]],
                                        ssem, add=True))
        for d in sds:
            d.wait()

    def step(i, carry):
        group(NBUF * i, NBUF)
        return carry

    for cbase in range(0, ANCHUNK, APH):
        nch = min(APH, ANCHUNK - cbase)
        pltpu.sync_copy(src_hbm.at[wid, pl.ds(cbase, nch), :],
                        sidx.at[pl.ds(0, nch), :])
        pltpu.sync_copy(dst_hbm.at[wid, pl.ds(cbase, nch), :],
                        didx.at[pl.ds(0, nch), :])
        lax.fori_loop(0, nch // NBUF, step, 0)
        if nch % NBUF:
            group(nch - nch % NBUF, nch % NBUF)

    plsc.subcore_barrier()

    @pl.when(s < NS - 1)
    def _():
        pltpu.sync_copy(accum.at[pl.ds(off, ZB), :],
                        out_hbm.at[c, pl.ds(off, ZB), :])

    @pl.when(s == NS - 1)
    def _():
        pltpu.sync_copy(accum.at[pl.ds((NS - 1) * ZB, ZL), :],
                        out_hbm.at[c, pl.ds((NS - 1) * ZB, ZL), :])


_agg = pl.kernel(
    _agg_body,
    out_type=jax.ShapeDtypeStruct((NC, N_NODES, D), jnp.float32),
    mesh=_MESH,
    scratch_types=[
        pltpu.VMEM((APH, AK), jnp.int32),
        pltpu.VMEM((APH, AK), jnp.int32),
        pltpu.VMEM((NBUF, AK, D), jnp.float32),
        pltpu.VMEM_SHARED((N_NODES, D), jnp.float32),
        pltpu.SemaphoreType.DMA,
    ] + [pltpu.SemaphoreType.DMA] * NBUF)


_HR = 80


def _counts_body(dst_hbm, zf_hbm, idt_hbm, cout_hbm, dflat, cl, idt, csh):
    c = lax.axis_index("c")
    s = lax.axis_index("s")
    wid = s * NC + c

    pltpu.sync_copy(zf_hbm.at[pl.ds(0, _HR), :], cl)
    pltpu.sync_copy(idt_hbm, idt)
    pltpu.sync_copy(dst_hbm.at[wid], dflat)

    @pl.when(s == 0)
    def _():
        pltpu.sync_copy(zf_hbm.at[pl.ds(0, _HR), :], csh)

    plsc.subcore_barrier()

    ones16 = jnp.ones((16,), jnp.float32)

    def hstep(k, carry):
        idx = dflat[pl.ds(k * 16, 16)]
        row = lax.shift_right_logical(idx, 7)
        lane = lax.bitwise_and(idx, 127)
        plsc.addupdate_scatter(cl, [row, lane], ones16)
        return carry

    lax.fori_loop(0, EW // 16, hstep, 0)
    pltpu.sync_copy(cl, csh.at[idt.at[0]], add=True)
    plsc.subcore_barrier()

    @pl.when(s == 0)
    def _():
        pltpu.sync_copy(csh, cout_hbm.at[c])


_counts = pl.kernel(
    _counts_body,
    out_type=jax.ShapeDtypeStruct((NC, _HR, D), jnp.float32),
    mesh=_MESH,
    compiler_params=pltpu.CompilerParams(needs_layout_passes=False),
    scratch_types=[
        pltpu.VMEM((EW,), jnp.int32),
        pltpu.VMEM((_HR, D), jnp.float32),
        pltpu.VMEM((1, _HR), jnp.int32),
        pltpu.VMEM_SHARED((_HR, D), jnp.float32),
    ])


_BR = 2000


def _dense_body(x_ref, wl_ref, wr_ref, b_ref, u_ref, v_ref):
    xb = x_ref[...]
    u_ref[...] = lax.dot_general(xb, wl_ref[...], _DN,
                                 preferred_element_type=jnp.float32,
                                 precision=_PREC)
    v_ref[...] = lax.dot_general(xb, wr_ref[...], _DN,
                                 preferred_element_type=jnp.float32,
                                 precision=_PREC) + b_ref[...]


def _recip_body(cnt_ref, r_ref):
    r_ref[...] = 1.0 / jnp.maximum(cnt_ref[0] + cnt_ref[1], 1.0)


def _combine_body(s_ref, rc_ref, v1_ref, wl_ref, wr_ref, b_ref,
                  u2_ref, v2_ref):
    h = jnp.maximum((s_ref[0] + s_ref[1]) * rc_ref[...] + v1_ref[...], 0.0)
    u2_ref[...] = lax.dot_general(h, wl_ref[...], _DN,
                                  preferred_element_type=jnp.float32,
                                  precision=_PREC)
    v2_ref[...] = lax.dot_general(h, wr_ref[...], _DN,
                                  preferred_element_type=jnp.float32,
                                  precision=_PREC) + b_ref[...]


def _final_body(s_ref, rc_ref, v2_ref, o_ref):
    o_ref[...] = (s_ref[0] + s_ref[1]) * rc_ref[...] + v2_ref[...]


_row_spec = pl.BlockSpec((_BR, D), lambda i: (i, 0))
_w_spec = pl.BlockSpec((D, D), lambda i: (0, 0))
_b_spec = pl.BlockSpec((1, D), lambda i: (0, 0))
_part_spec = pl.BlockSpec((NC, _BR, D), lambda i: (0, i, 0))
_rc_spec = pl.BlockSpec((_BR, 1), lambda i: (i, 0))
_GRID = (N_NODES // _BR,)
_f32 = functools.partial(jax.ShapeDtypeStruct, dtype=jnp.float32)

_dense = pl.pallas_call(
    _dense_body, grid=_GRID,
    in_specs=[_row_spec, _w_spec, _w_spec, _b_spec],
    out_specs=[_row_spec, _row_spec],
    out_shape=[_f32((N_NODES, D)), _f32((N_NODES, D))])

_recip = pl.pallas_call(
    _recip_body,
    in_specs=[pl.BlockSpec((NC, _HR, D), lambda: (0, 0, 0))],
    out_specs=pl.BlockSpec((_HR, D), lambda: (0, 0)),
    out_shape=_f32((_HR, D)))

_combine = pl.pallas_call(
    _combine_body, grid=_GRID,
    in_specs=[_part_spec, _rc_spec, _row_spec, _w_spec, _w_spec, _b_spec],
    out_specs=[_row_spec, _row_spec],
    out_shape=[_f32((N_NODES, D)), _f32((N_NODES, D))])

_final = pl.pallas_call(
    _final_body, grid=_GRID,
    in_specs=[_part_spec, _rc_spec, _row_spec],
    out_specs=_row_spec,
    out_shape=_f32((N_NODES, D)))


def kernel(x, edge_index, Wl1, bl1, Wr1, Wl2, bl2, Wr2):
    src = edge_index[0].astype(jnp.int32)
    dst = edge_index[1].astype(jnp.int32)
    src3a = src.reshape(NW, ANCHUNK, AK)
    dst3a = dst.reshape(NW, ANCHUNK, AK)
    dst2 = dst.reshape(NW, EW)
    zf = jnp.zeros((ZL, D), jnp.float32)
    idt = jnp.arange(_HR, dtype=jnp.int32).reshape(1, _HR)

    u1, v1 = _dense(x, Wl1, Wr1, bl1.reshape(1, D))
    cnt = _counts(dst2, zf, idt)
    rc = _recip(cnt).reshape(_HR * D)[:N_NODES].reshape(N_NODES, 1)
    s1 = _agg(u1, src3a, dst3a, zf)
    u2, v2 = _combine(s1, rc, v1, Wl2, Wr2, bl2.reshape(1, D))
    s2 = _agg(u2, src3a, dst3a, zf)
    return _final(s2, rc, v2)


# speedup vs baseline: 1.0590x; 1.0067x over previous
import functools

import jax
import jax.numpy as jnp
from jax import lax
from jax.experimental import pallas as pl
from jax.experimental.pallas import tpu as pltpu
from jax.experimental.pallas import tpu_sc as plsc

N_NODES = 10000
D = 128
N_EDGES = 320000

NC = 2
NS = 16
NW = NC * NS
EW = N_EDGES // NW
AK = 40
ANCHUNK = EW // AK
APH = 48
NBUF = 6
ZB = 624
ZL = N_NODES - (NS - 1) * ZB

_DN = (((1,), (1,)), ((), ()))
_PREC = None


_MESH = plsc.VectorSubcoreMesh(core_axis_name="c", subcore_axis_name="s")


def _agg_body(u_hbm, src_hbm, dst_hbm, zf_hbm, out_hbm,
              sidx, didx, rows, accum, ssem, *gsems):
    c = lax.axis_index("c")
    s = lax.axis_index("s")
    wid = s * NC + c

    off = pl.multiple_of(s * ZB, 8)

    @pl.when(s < NS - 1)
    def _():
        pltpu.sync_copy(zf_hbm.at[pl.ds(0, ZB), :],
                        accum.at[pl.ds(off, ZB), :])

    @pl.when(s == NS - 1)
    def _():
        pltpu.sync_copy(zf_hbm, accum.at[pl.ds((NS - 1) * ZB, ZL), :])

    plsc.subcore_barrier()

    def group(l0, nb):
        gds = [pltpu.async_copy(u_hbm.at[sidx.at[l0 + b]], rows.at[b],
                                gsems[b])
               for b in range(nb)]
        sds = []
        for b in range(nb):
            gds[b].wait()
            sds.append(pltpu.async_copy(rows.at[b], accum.at[didx.at[l0 + b]],
                                        ssem, add=True))
        for d in sds:
            d.wait()

    def step(i, carry):
        group(NBUF * i, NBUF)
        return carry

    for cbase in range(0, ANCHUNK, APH):
        nch = min(APH, ANCHUNK - cbase)
        pltpu.sync_copy(src_hbm.at[wid, pl.ds(cbase, nch), :],
                        sidx.at[pl.ds(0, nch), :])
        pltpu.sync_copy(dst_hbm.at[wid, pl.ds(cbase, nch), :],
                        didx.at[pl.ds(0, nch), :])
        lax.fori_loop(0, nch // NBUF, step, 0)
        if nch % NBUF:
            group(nch - nch % NBUF, nch % NBUF)

    plsc.subcore_barrier()

    @pl.when(s < NS - 1)
    def _():
        pltpu.sync_copy(accum.at[pl.ds(off, ZB), :],
                        out_hbm.at[c, pl.ds(off, ZB), :])

    @pl.when(s == NS - 1)
    def _():
        pltpu.sync_copy(accum.at[pl.ds((NS - 1) * ZB, ZL), :],
                        out_hbm.at[c, pl.ds((NS - 1) * ZB, ZL), :])


_agg = pl.kernel(
    _agg_body,
    out_type=jax.ShapeDtypeStruct((NC, N_NODES, D), jnp.float32),
    mesh=_MESH,
    scratch_types=[
        pltpu.VMEM((APH, AK), jnp.int32),
        pltpu.VMEM((APH, AK), jnp.int32),
        pltpu.VMEM((NBUF, AK, D), jnp.float32),
        pltpu.VMEM_SHARED((N_NODES, D), jnp.float32),
        pltpu.SemaphoreType.DMA,
    ] + [pltpu.SemaphoreType.DMA] * NBUF)


_HR = 80


def _counts_body(dst_hbm, zf_hbm, idt_hbm, cout_hbm, dflat, cl, idt, csh):
    c = lax.axis_index("c")
    s = lax.axis_index("s")
    wid = s * NC + c

    pltpu.sync_copy(zf_hbm.at[pl.ds(0, _HR), :], cl)
    pltpu.sync_copy(idt_hbm, idt)
    pltpu.sync_copy(dst_hbm.at[wid], dflat)

    @pl.when(s == 0)
    def _():
        pltpu.sync_copy(zf_hbm.at[pl.ds(0, _HR), :], csh)

    plsc.subcore_barrier()

    ones16 = jnp.ones((16,), jnp.float32)

    def hstep(k, carry):
        idx = dflat[pl.ds(k * 16, 16)]
        row = lax.shift_right_logical(idx, 7)
        lane = lax.bitwise_and(idx, 127)
        plsc.addupdate_scatter(cl, [row, lane], ones16)
        return carry

    lax.fori_loop(0, EW // 16, hstep, 0)
    pltpu.sync_copy(cl, csh.at[idt.at[0]], add=True)
    plsc.subcore_barrier()

    @pl.when(s == 0)
    def _():
        pltpu.sync_copy(csh, cout_hbm.at[c])


_counts = pl.kernel(
    _counts_body,
    out_type=jax.ShapeDtypeStruct((NC, _HR, D), jnp.float32),
    mesh=_MESH,
    compiler_params=pltpu.CompilerParams(needs_layout_passes=False),
    scratch_types=[
        pltpu.VMEM((EW,), jnp.int32),
        pltpu.VMEM((_HR, D), jnp.float32),
        pltpu.VMEM((1, _HR), jnp.int32),
        pltpu.VMEM_SHARED((_HR, D), jnp.float32),
    ])


_BR = 5000


def _dense_body(x_ref, wl_ref, wr_ref, b_ref, u_ref, v_ref):
    xb = x_ref[...]
    u_ref[...] = lax.dot_general(xb, wl_ref[...], _DN,
                                 preferred_element_type=jnp.float32,
                                 precision=_PREC)
    v_ref[...] = lax.dot_general(xb, wr_ref[...], _DN,
                                 preferred_element_type=jnp.float32,
                                 precision=_PREC) + b_ref[...]


def _recip_body(cnt_ref, r_ref):
    r_ref[...] = 1.0 / jnp.maximum(cnt_ref[0] + cnt_ref[1], 1.0)


def _combine_body(s_ref, rc_ref, v1_ref, wl_ref, wr_ref, b_ref,
                  u2_ref, v2_ref):
    h = jnp.maximum((s_ref[0] + s_ref[1]) * rc_ref[...] + v1_ref[...], 0.0)
    u2_ref[...] = lax.dot_general(h, wl_ref[...], _DN,
                                  preferred_element_type=jnp.float32,
                                  precision=_PREC)
    v2_ref[...] = lax.dot_general(h, wr_ref[...], _DN,
                                  preferred_element_type=jnp.float32,
                                  precision=_PREC) + b_ref[...]


def _final_body(s_ref, rc_ref, v2_ref, o_ref):
    o_ref[...] = (s_ref[0] + s_ref[1]) * rc_ref[...] + v2_ref[...]


_row_spec = pl.BlockSpec((_BR, D), lambda i: (i, 0))
_w_spec = pl.BlockSpec((D, D), lambda i: (0, 0))
_b_spec = pl.BlockSpec((1, D), lambda i: (0, 0))
_part_spec = pl.BlockSpec((NC, _BR, D), lambda i: (0, i, 0))
_rc_spec = pl.BlockSpec((_BR, 1), lambda i: (i, 0))
_GRID = (N_NODES // _BR,)
_f32 = functools.partial(jax.ShapeDtypeStruct, dtype=jnp.float32)

_dense = pl.pallas_call(
    _dense_body, grid=_GRID,
    in_specs=[_row_spec, _w_spec, _w_spec, _b_spec],
    out_specs=[_row_spec, _row_spec],
    out_shape=[_f32((N_NODES, D)), _f32((N_NODES, D))])

_recip = pl.pallas_call(
    _recip_body,
    in_specs=[pl.BlockSpec((NC, _HR, D), lambda: (0, 0, 0))],
    out_specs=pl.BlockSpec((_HR, D), lambda: (0, 0)),
    out_shape=_f32((_HR, D)))

_combine = pl.pallas_call(
    _combine_body, grid=_GRID,
    in_specs=[_part_spec, _rc_spec, _row_spec, _w_spec, _w_spec, _b_spec],
    out_specs=[_row_spec, _row_spec],
    out_shape=[_f32((N_NODES, D)), _f32((N_NODES, D))])

_final = pl.pallas_call(
    _final_body, grid=_GRID,
    in_specs=[_part_spec, _rc_spec, _row_spec],
    out_specs=_row_spec,
    out_shape=_f32((N_NODES, D)))


def kernel(x, edge_index, Wl1, bl1, Wr1, Wl2, bl2, Wr2):
    src = edge_index[0].astype(jnp.int32)
    dst = edge_index[1].astype(jnp.int32)
    src3a = src.reshape(NW, ANCHUNK, AK)
    dst3a = dst.reshape(NW, ANCHUNK, AK)
    dst2 = dst.reshape(NW, EW)
    zf = jnp.zeros((ZL, D), jnp.float32)
    idt = jnp.arange(_HR, dtype=jnp.int32).reshape(1, _HR)

    u1, v1 = _dense(x, Wl1, Wr1, bl1.reshape(1, D))
    cnt = _counts(dst2, zf, idt)
    rc = _recip(cnt).reshape(_HR * D)[:N_NODES].reshape(N_NODES, 1)
    s1 = _agg(u1, src3a, dst3a, zf)
    u2, v2 = _combine(s1, rc, v1, Wl2, Wr2, bl2.reshape(1, D))
    s2 = _agg(u2, src3a, dst3a, zf)
    return _final(s2, rc, v2)
